# baseline (device time: 60691 ns/iter reference)
import jax
import jax.numpy as jnp
from jax import lax
from jax.experimental import pallas as pl
from jax.experimental.pallas import tpu as pltpu

N_DEV = 4
N_TOK = 2048
D_MODEL = 1024
N_EXP = 32
EXP_PER_DEV = N_EXP // N_DEV
CAPACITY = 51
CAP_PAD = 64
ROWS = EXP_PER_DEV * CAP_PAD
HALF = ROWS // 2

MINE, FROM_L, FROM_R, DIAG = 0, ROWS, 2 * ROWS, 3 * ROWS


def _moe_body(x_ref, disp_ref, unperm_ref, w_hbm_ref, out_ref,
              y_all, xb, xg, w_buf, out_acc, w_sems,
              s_r, s_l, r_l, r_r, s_fwd, r_diag, out_sem):
    my_pos = lax.axis_index("i")
    left = (my_pos - 1) % N_DEV
    right = (my_pos + 1) % N_DEV

    barrier_sem = pltpu.get_barrier_semaphore()
    for nbr in [left, right]:
        pl.semaphore_signal(
            barrier_sem, inc=1,
            device_id=(nbr,), device_id_type=pl.DeviceIdType.MESH,
        )
    pl.semaphore_wait(barrier_sem, 2)

    def w_dma(e, slot):
        return pltpu.make_async_copy(
            w_hbm_ref.at[e], w_buf.at[slot], w_sems.at[slot]
        )

    def chunk(base, e):
        return y_all.at[pl.ds(base + e * CAP_PAD, CAP_PAD)]

    def rdma(src, dst, ssem, rsem, dev):
        return pltpu.make_async_remote_copy(
            src_ref=src, dst_ref=dst, send_sem=ssem, recv_sem=rsem,
            device_id=(dev,), device_id_type=pl.DeviceIdType.MESH,
        )

    with jax.named_scope("dispatch"):
        w_dma(0, 0).start()
        xb[:, :] = x_ref[:, :].astype(jnp.bfloat16)
        xg[:, :] = jnp.dot(
            disp_ref[:, :], xb[:, :], preferred_element_type=jnp.float32
        ).astype(jnp.bfloat16)
    with jax.named_scope("experts"):
        for e in range(EXP_PER_DEV):
            slot = e % 2
            w_dma(e, slot).wait()
            if e + 1 < EXP_PER_DEV:
                w_dma(e + 1, (e + 1) % 2).start()
            y_e = jnp.dot(
                xg[e * CAP_PAD:(e + 1) * CAP_PAD, :],
                w_buf[slot].astype(jnp.bfloat16),
                preferred_element_type=jnp.float32,
            ).astype(jnp.bfloat16)
            y_all[MINE + e * CAP_PAD:MINE + (e + 1) * CAP_PAD, :] = y_e
            rdma(chunk(MINE, e), chunk(FROM_L, e),
                 s_r.at[e], r_l.at[e], right).start()
            rdma(chunk(MINE, e), chunk(FROM_R, e),
                 s_l.at[e], r_r.at[e], left).start()

    with jax.named_scope("p1_wait_l"):
        for e in range(EXP_PER_DEV):
            rdma(chunk(FROM_L, e), chunk(FROM_L, e),
                 s_r.at[e], r_l.at[e], left).wait_recv()
    fwd_r = rdma(y_all.at[pl.ds(FROM_L, HALF)],
                 y_all.at[pl.ds(DIAG, HALF)],
                 s_fwd.at[0], r_diag.at[0], right)
    fwd_r.start()

    with jax.named_scope("p1_wait_r"):
        for e in range(EXP_PER_DEV):
            rdma(chunk(FROM_R, e), chunk(FROM_R, e),
                 s_l.at[e], r_r.at[e], right).wait_recv()
    fwd_l = rdma(y_all.at[pl.ds(FROM_R + HALF, HALF)],
                 y_all.at[pl.ds(DIAG + HALF, HALF)],
                 s_fwd.at[1], r_diag.at[1], left)
    fwd_l.start()

    with jax.named_scope("unperm012"):
        out_acc[:, :] = jnp.dot(
            unperm_ref[:, :3 * ROWS], y_all[:3 * ROWS, :],
            preferred_element_type=jnp.float32,
        ).astype(jnp.bfloat16)

    with jax.named_scope("p2_wait_diag"):
        rdma(y_all.at[pl.ds(DIAG, HALF)], y_all.at[pl.ds(DIAG, HALF)],
             s_fwd.at[0], r_diag.at[0], left).wait_recv()
        rdma(y_all.at[pl.ds(DIAG + HALF, HALF)],
             y_all.at[pl.ds(DIAG + HALF, HALF)],
             s_fwd.at[1], r_diag.at[1], right).wait_recv()
    with jax.named_scope("unperm3"):
        out_acc[:, :] += jnp.dot(
            unperm_ref[:, 3 * ROWS:], y_all[3 * ROWS:, :],
            preferred_element_type=jnp.float32,
        ).astype(jnp.bfloat16)

    with jax.named_scope("out_dma"):
        out_dma = pltpu.make_async_copy(out_acc, out_ref, out_sem)
        out_dma.start()
        out_dma.wait()

    with jax.named_scope("drain"):
        for e in range(EXP_PER_DEV):
            rdma(chunk(MINE, e), chunk(FROM_L, e),
                 s_r.at[e], r_l.at[e], right).wait_send()
            rdma(chunk(MINE, e), chunk(FROM_R, e),
                 s_l.at[e], r_r.at[e], left).wait_send()
        fwd_r.wait_send()
        fwd_l.wait_send()


def _moe_pallas(x, disp, unperm, expert_W):
    return pl.pallas_call(
        _moe_body,
        out_shape=jax.ShapeDtypeStruct((N_TOK, D_MODEL), jnp.bfloat16),
        in_specs=[
            pl.BlockSpec(memory_space=pltpu.VMEM),
            pl.BlockSpec(memory_space=pltpu.VMEM),
            pl.BlockSpec(memory_space=pltpu.VMEM),
            pl.BlockSpec(memory_space=pl.ANY),
        ],
        out_specs=pl.BlockSpec(memory_space=pl.ANY),
        scratch_shapes=[
            pltpu.VMEM((N_DEV * ROWS, D_MODEL), jnp.bfloat16),
            pltpu.VMEM((N_TOK, D_MODEL), jnp.bfloat16),
            pltpu.VMEM((ROWS, D_MODEL), jnp.bfloat16),
            pltpu.VMEM((2, D_MODEL, D_MODEL), jnp.float32),
            pltpu.VMEM((N_TOK, D_MODEL), jnp.bfloat16),
            pltpu.SemaphoreType.DMA((2,)),
            pltpu.SemaphoreType.DMA((EXP_PER_DEV,)),
            pltpu.SemaphoreType.DMA((EXP_PER_DEV,)),
            pltpu.SemaphoreType.DMA((EXP_PER_DEV,)),
            pltpu.SemaphoreType.DMA((EXP_PER_DEV,)),
            pltpu.SemaphoreType.DMA((2,)),
            pltpu.SemaphoreType.DMA((2,)),
            pltpu.SemaphoreType.DMA(()),
        ],
        compiler_params=pltpu.CompilerParams(
            collective_id=0, vmem_limit_bytes=60 * 1024 * 1024
        ),
    )(x, disp, unperm, expert_W)


def kernel(x, router_W, route_idx, expert_W):
    del router_W
    my_i = lax.axis_index("i")

    e = route_idx[:, 0]
    B, T = 64, 32
    oh = (e[:, None] == jnp.arange(N_EXP)[None, :]).astype(jnp.float32)
    oh3 = oh.reshape(B, T, N_EXP)
    within = jnp.einsum(
        "ts,bse->bte", jnp.tril(jnp.ones((T, T), jnp.float32), -1), oh3
    )
    prefix = jnp.dot(
        jnp.tril(jnp.ones((B, B), jnp.float32), -1), oh3.sum(axis=1)
    )
    rank3 = within + prefix[:, None, :]
    rank = jnp.sum(rank3 * oh3, axis=2).reshape(N_TOK).astype(jnp.int32)
    kept = rank < CAPACITY
    src_row = e * CAP_PAD + rank

    loc_rows = my_i * ROWS + jnp.arange(ROWS)
    disp = jnp.where(
        (src_row[None, :] == loc_rows[:, None]) & kept[None, :], 1.0, 0.0
    ).astype(jnp.bfloat16)

    dev_order = jnp.stack(
        [my_i, (my_i - 1) % N_DEV, (my_i + 1) % N_DEV, (my_i + 2) % N_DEV]
    )
    col_g = (dev_order[:, None] * ROWS + jnp.arange(ROWS)[None, :]).reshape(-1)
    unperm = jnp.where(
        (src_row[:, None] == col_g[None, :]) & kept[:, None], 1.0, 0.0
    ).astype(jnp.bfloat16)

    return _moe_pallas(x, disp, unperm, expert_W)
